# baseline (speedup 1.0000x reference)
import functools

import jax
import jax.numpy as jnp
from jax import lax
from jax.experimental import pallas as pl
from jax.experimental.pallas import tpu as pltpu
from jax.experimental.pallas import tpu_sc as plsc

M = 8
K = 10
L = 5
C = 3
H = 64
W = 64
N = 32
DOUT = 64
NCLS = 10
NWORDS = 2 ** K
PAD = L // 2
HW = H * W
HP = H // 2
WP = W // 2


def _stage1_body(off_ref, thr_ref, xp_ref, idx_ref, conf_ref):
    m = pl.program_id(0)
    word = jnp.zeros((N, H, W), jnp.int32)
    conf = jnp.full((N, H, W), 0.25, jnp.float32)
    for k in range(K):
        c1k = off_ref[m, k, 0]
        dy1k = off_ref[m, k, 1]
        dx1k = off_ref[m, k, 2]
        c2k = off_ref[m, k, 3]
        dy2k = off_ref[m, k, 4]
        dx2k = off_ref[m, k, 5]
        v1 = xp_ref[:, c1k, pl.ds(dy1k, H), :]
        v2 = xp_ref[:, c2k, pl.ds(dy2k, H), :]
        p1 = pltpu.roll(v1, 68 - dx1k, axis=2)[:, :, :W]
        p2 = pltpu.roll(v2, 68 - dx2k, axis=2)[:, :, :W]
        z = (p1 - p2) - thr_ref[m, k]
        bit = z > 0.0
        word = word + jnp.where(bit, jnp.int32(1 << k), jnp.int32(0))
        s = 1.0 / (1.0 + jnp.exp(-z))
        conf = conf * jnp.where(bit, s, 1.0 - s)
    idx_ref[0] = (word + m * NWORDS).reshape(N, HW)
    conf_ref[0] = conf.reshape(N, HW)


def _stage1(xp, offs, thr):
    return pl.pallas_call(
        _stage1_body,
        grid=(M,),
        in_specs=[
            pl.BlockSpec(memory_space=pltpu.SMEM),
            pl.BlockSpec(memory_space=pltpu.SMEM),
            pl.BlockSpec((N, C, H + 2 * PAD, W + 2 * PAD),
                         lambda m: (0, 0, 0, 0)),
        ],
        out_specs=[
            pl.BlockSpec((1, N, HW), lambda m: (m, 0, 0)),
            pl.BlockSpec((1, N, HW), lambda m: (m, 0, 0)),
        ],
        out_shape=[
            jax.ShapeDtypeStruct((M, N, HW), jnp.int32),
            jax.ShapeDtypeStruct((M, N, HW), jnp.float32),
        ],
    )(offs, thr, xp)


def _sc_body(idx_hbm, conf_hbm, table_hbm, out_hbm,
             idx_v, conf_v, gbuf, acc, gsem0, gsem1, lsem, osem):
    cid = lax.axis_index("c")
    sid = lax.axis_index("s")
    n = sid * 2 + cid

    for m in range(M):
        pltpu.async_copy(idx_hbm.at[m, n], idx_v.at[m], lsem)
        pltpu.async_copy(conf_hbm.at[m, n], conf_v.at[m], lsem)
    for m in range(M):
        pltpu.make_async_copy(idx_hbm.at[m, n], idx_v.at[m], lsem).wait()
        pltpu.make_async_copy(conf_hbm.at[m, n], conf_v.at[m], lsem).wait()

    zero = jnp.zeros((16,), jnp.float32)
    for jj in range(WP):
        for q in range(4):
            acc[jj, pl.ds(16 * q, 16)] = zero

    def _issue(c, buf, sem):
        px0 = (c // 2) * W + lax.rem(c, 2) * 32
        for m in range(M):
            pltpu.async_copy(
                table_hbm.at[idx_v.at[m, pl.ds(px0, 32)]],
                gbuf.at[buf, m], sem)

    def _wait(c, buf, sem):
        px0 = (c // 2) * W + lax.rem(c, 2) * 32
        for m in range(M):
            pltpu.make_async_copy(
                table_hbm.at[idx_v.at[m, pl.ds(px0, 32)]],
                gbuf.at[buf, m], sem).wait()

    def _compute(c, buf):
        px0 = (c // 2) * W + lax.rem(c, 2) * 32
        jbase = lax.rem(c, 2) * 16
        cvecs = [[conf_v[m, pl.ds(px0 + 16 * half, 16)] for half in range(2)]
                 for m in range(M)]
        for p in range(16):
            a = [None] * 4
            for m in range(M):
                for b in range(2):
                    lane = 2 * p + b
                    cv = jnp.full((16,), cvecs[m][lane // 16][lane % 16],
                                  jnp.float32)
                    for q in range(4):
                        r = gbuf[buf, m, 2 * p + b, pl.ds(16 * q, 16)]
                        t = cv * r
                        a[q] = t if a[q] is None else a[q] + t
            for q in range(4):
                plsc.addupdate(acc.at[jbase + p, pl.ds(16 * q, 16)], a[q])

    _issue(0, 0, gsem0)

    @pl.loop(0, 2 * HP)
    def _pair(t):
        c0 = 2 * t
        c1 = 2 * t + 1
        i = t // 2
        hh = lax.rem(t, 2)
        _issue(c1, 1, gsem1)
        _wait(c0, 0, gsem0)
        _compute(c0, 0)

        @pl.when(t < 2 * HP - 1)
        def _():
            _issue(c0 + 2, 0, gsem0)

        _wait(c1, 1, gsem1)
        _compute(c1, 1)

        @pl.when(hh == 1)
        def _():
            pltpu.sync_copy(acc, out_hbm.at[n, i])
            for jj in range(WP):
                for q in range(4):
                    acc[jj, pl.ds(16 * q, 16)] = zero


@functools.cache
def _sc_gather():
    mesh = plsc.VectorSubcoreMesh(core_axis_name="c", subcore_axis_name="s")
    return pl.kernel(
        _sc_body,
        out_type=jax.ShapeDtypeStruct((N, HP, WP, DOUT), jnp.float32),
        mesh=mesh,
        scratch_types=[
            pltpu.VMEM((M, HW), jnp.int32),
            pltpu.VMEM((M, HW), jnp.float32),
            pltpu.VMEM((2, M, 32, DOUT), jnp.float32),
            pltpu.VMEM((WP, DOUT), jnp.float32),
            pltpu.SemaphoreType.DMA,
            pltpu.SemaphoreType.DMA,
            pltpu.SemaphoreType.DMA,
            pltpu.SemaphoreType.DMA,
        ],
        compiler_params=pltpu.CompilerParams(use_tc_tiling_on_sc=False),
    )


def _mm_body(a_ref, bt_ref, o_ref):
    o_ref[...] = jax.lax.dot_general(
        a_ref[...], bt_ref[...], (((1,), (1,)), ((), ())),
        preferred_element_type=jnp.float32)


def _stage3(flat, wt_t):
    return pl.pallas_call(
        _mm_body,
        out_shape=jax.ShapeDtypeStruct((N, NCLS), jnp.float32),
    )(flat, wt_t)


def kernel(x, thresholds, table, W_pred, b_pred, dy1, dx1, c1, dy2, dx2, c2):
    xp = jnp.pad(x, ((0, 0), (0, 0), (PAD, PAD), (PAD, PAD)))
    offs = jnp.stack([c1, dy1, dx1, c2, dy2, dx2], axis=-1).astype(jnp.int32)
    idx, conf = _stage1(xp, offs, thresholds)
    pooled = _sc_gather()(idx, conf, table)
    flat = pooled.reshape(N, HP * WP * DOUT)
    wt_t = W_pred.reshape(DOUT, HP * WP, NCLS).transpose(2, 1, 0).reshape(
        NCLS, HP * WP * DOUT)
    return _stage3(flat, wt_t) + b_pred


# speedup vs baseline: 1.1019x; 1.1019x over previous
import functools

import jax
import jax.numpy as jnp
from jax import lax
from jax.experimental import pallas as pl
from jax.experimental.pallas import tpu as pltpu
from jax.experimental.pallas import tpu_sc as plsc

M = 8
K = 10
L = 5
C = 3
H = 64
W = 64
N = 32
DOUT = 64
NCLS = 10
NWORDS = 2 ** K
PAD = L // 2
HW = H * W
HP = H // 2
WP = W // 2


def _stage1_body(nb, off_ref, thr_ref, xp_ref, idx_ref, conf_ref):
    m = pl.program_id(0)
    word = jnp.zeros((nb, H, W), jnp.int32)
    conf = jnp.full((nb, H, W), 0.25, jnp.float32)
    for k in range(K):
        c1k = off_ref[m, k, 0]
        dy1k = off_ref[m, k, 1]
        dx1k = off_ref[m, k, 2]
        c2k = off_ref[m, k, 3]
        dy2k = off_ref[m, k, 4]
        dx2k = off_ref[m, k, 5]
        v1 = xp_ref[:, c1k, pl.ds(dy1k, H), :]
        v2 = xp_ref[:, c2k, pl.ds(dy2k, H), :]
        p1 = pltpu.roll(v1, 68 - dx1k, axis=2)[:, :, :W]
        p2 = pltpu.roll(v2, 68 - dx2k, axis=2)[:, :, :W]
        z = (p1 - p2) - thr_ref[m, k]
        bit = z > 0.0
        word = word + jnp.where(bit, jnp.int32(1 << k), jnp.int32(0))
        s = 1.0 / (1.0 + jnp.exp(-z))
        conf = conf * jnp.where(bit, s, 1.0 - s)
    idx_ref[0] = (word + m * NWORDS).reshape(nb, HW)
    conf_ref[0] = conf.reshape(nb, HW)


def _stage1(xp, offs, thr):
    nb = xp.shape[0]
    return pl.pallas_call(
        functools.partial(_stage1_body, nb),
        grid=(M,),
        in_specs=[
            pl.BlockSpec(memory_space=pltpu.SMEM),
            pl.BlockSpec(memory_space=pltpu.SMEM),
            pl.BlockSpec((nb, C, H + 2 * PAD, W + 2 * PAD),
                         lambda m: (0, 0, 0, 0)),
        ],
        out_specs=[
            pl.BlockSpec((1, nb, HW), lambda m: (m, 0, 0)),
            pl.BlockSpec((1, nb, HW), lambda m: (m, 0, 0)),
        ],
        out_shape=[
            jax.ShapeDtypeStruct((M, nb, HW), jnp.int32),
            jax.ShapeDtypeStruct((M, nb, HW), jnp.float32),
        ],
    )(offs, thr, xp)


def _sc_body(idx_hbm, conf_hbm, table_hbm, out_hbm,
             idx_v, conf_v, gbuf, acc, gsem0, gsem1, lsem, osem):
    cid = lax.axis_index("c")
    sid = lax.axis_index("s")
    wid = sid * 2 + cid
    n = wid // 2
    rh = lax.rem(wid, 2)

    for m in range(M):
        pltpu.async_copy(idx_hbm.at[m, n], idx_v.at[m], lsem)
        pltpu.async_copy(conf_hbm.at[m, n], conf_v.at[m], lsem)
    for m in range(M):
        pltpu.make_async_copy(idx_hbm.at[m, n], idx_v.at[m], lsem).wait()
        pltpu.make_async_copy(conf_hbm.at[m, n], conf_v.at[m], lsem).wait()

    zero = jnp.zeros((16,), jnp.float32)
    for jj in range(WP):
        for q in range(4):
            acc[jj, pl.ds(16 * q, 16)] = zero

    def _issue(c, buf, sem):
        px0 = (c // 2) * W + lax.rem(c, 2) * 32
        for m in range(M):
            pltpu.async_copy(
                table_hbm.at[idx_v.at[m, pl.ds(px0, 32)]],
                gbuf.at[buf, m], sem)

    def _wait(c, buf, sem):
        px0 = (c // 2) * W + lax.rem(c, 2) * 32
        for m in range(M):
            pltpu.make_async_copy(
                table_hbm.at[idx_v.at[m, pl.ds(px0, 32)]],
                gbuf.at[buf, m], sem).wait()

    def _compute(c, buf):
        px0 = (c // 2) * W + lax.rem(c, 2) * 32
        jbase = lax.rem(c, 2) * 16
        cvecs = [[conf_v[m, pl.ds(px0 + 16 * half, 16)] for half in range(2)]
                 for m in range(M)]
        for p in range(16):
            a = [None] * 4
            for m in range(M):
                for b in range(2):
                    lane = 2 * p + b
                    cv = jnp.full((16,), cvecs[m][lane // 16][lane % 16],
                                  jnp.float32)
                    for q in range(4):
                        r = gbuf[buf, m, 2 * p + b, pl.ds(16 * q, 16)]
                        t = cv * r
                        a[q] = t if a[q] is None else a[q] + t
            for q in range(4):
                plsc.addupdate(acc.at[jbase + p, pl.ds(16 * q, 16)], a[q])

    t0 = rh * HP
    _issue(2 * t0, 0, gsem0)

    @pl.loop(t0, t0 + HP)
    def _pair(t):
        c0 = 2 * t
        c1 = 2 * t + 1
        i = t // 2
        hh = lax.rem(t, 2)
        _issue(c1, 1, gsem1)
        _wait(c0, 0, gsem0)
        _compute(c0, 0)

        @pl.when(t < t0 + HP - 1)
        def _():
            _issue(c0 + 2, 0, gsem0)

        _wait(c1, 1, gsem1)
        _compute(c1, 1)

        @pl.when(hh == 1)
        def _():
            pltpu.sync_copy(acc, out_hbm.at[n, i])
            for jj in range(WP):
                for q in range(4):
                    acc[jj, pl.ds(16 * q, 16)] = zero


@functools.cache
def _sc_gather():
    mesh = plsc.VectorSubcoreMesh(core_axis_name="c", subcore_axis_name="s")
    return pl.kernel(
        _sc_body,
        out_type=jax.ShapeDtypeStruct((N // 2, HP, WP, DOUT), jnp.float32),
        mesh=mesh,
        scratch_types=[
            pltpu.VMEM((M, HW), jnp.int32),
            pltpu.VMEM((M, HW), jnp.float32),
            pltpu.VMEM((2, M, 32, DOUT), jnp.float32),
            pltpu.VMEM((WP, DOUT), jnp.float32),
            pltpu.SemaphoreType.DMA,
            pltpu.SemaphoreType.DMA,
            pltpu.SemaphoreType.DMA,
            pltpu.SemaphoreType.DMA,
        ],
        compiler_params=pltpu.CompilerParams(use_tc_tiling_on_sc=False),
    )


def _mm_body(a_ref, bt_ref, o_ref):
    o_ref[...] = jax.lax.dot_general(
        a_ref[...], bt_ref[...], (((1,), (1,)), ((), ())),
        preferred_element_type=jnp.float32)


def _stage3(flat, wt_t):
    return pl.pallas_call(
        _mm_body,
        out_shape=jax.ShapeDtypeStruct((N, NCLS), jnp.float32),
    )(flat, wt_t)


def kernel(x, thresholds, table, W_pred, b_pred, dy1, dx1, c1, dy2, dx2, c2):
    xp = jnp.pad(x, ((0, 0), (0, 0), (PAD, PAD), (PAD, PAD)))
    offs = jnp.stack([c1, dy1, dx1, c2, dy2, dx2], axis=-1).astype(jnp.int32)
    halves = []
    for h0 in range(2):
        xph = xp[h0 * (N // 2):(h0 + 1) * (N // 2)]
        idx, conf = _stage1(xph, offs, thresholds)
        halves.append(_sc_gather()(idx, conf, table))
    pooled = jnp.concatenate(halves, axis=0)
    flat = pooled.reshape(N, HP * WP * DOUT)
    wt_t = W_pred.reshape(DOUT, HP * WP, NCLS).transpose(2, 1, 0).reshape(
        NCLS, HP * WP * DOUT)
    return _stage3(flat, wt_t) + b_pred


# speedup vs baseline: 1.5983x; 1.4505x over previous
import functools

import numpy as np

import jax
import jax.numpy as jnp
from jax import lax
from jax.experimental import pallas as pl
from jax.experimental.pallas import tpu as pltpu
from jax.experimental.pallas import tpu_sc as plsc

M = 8
K = 10
L = 5
C = 3
H = 64
W = 64
N = 32
DOUT = 64
NCLS = 10
NWORDS = 2 ** K
PAD = L // 2
HW = H * W
HP = H // 2
WP = W // 2


def _stage1_body(nb, off_ref, thr_ref, xp_ref, idx_ref, conf_ref):
    m = pl.program_id(0)
    word = jnp.zeros((nb, H, W), jnp.int32)
    conf = jnp.full((nb, H, W), 0.25, jnp.float32)
    for k in range(K):
        c1k = off_ref[m, k, 0]
        dy1k = off_ref[m, k, 1]
        dx1k = off_ref[m, k, 2]
        c2k = off_ref[m, k, 3]
        dy2k = off_ref[m, k, 4]
        dx2k = off_ref[m, k, 5]
        v1 = xp_ref[:, c1k, pl.ds(dy1k, H), :]
        v2 = xp_ref[:, c2k, pl.ds(dy2k, H), :]
        p1 = pltpu.roll(v1, 68 - dx1k, axis=2)[:, :, :W]
        p2 = pltpu.roll(v2, 68 - dx2k, axis=2)[:, :, :W]
        z = (p1 - p2) - thr_ref[m, k]
        bit = z > 0.0
        word = word + jnp.where(bit, jnp.int32(1 << k), jnp.int32(0))
        s = 1.0 / (1.0 + jnp.exp(-z))
        conf = conf * jnp.where(bit, s, 1.0 - s)
    idx_ref[0] = (word + m * NWORDS).reshape(nb, HW)
    conf_ref[0] = conf.reshape(nb, HW)


def _stage1(xp, offs, thr):
    nb = xp.shape[0]
    return pl.pallas_call(
        functools.partial(_stage1_body, nb),
        grid=(M,),
        in_specs=[
            pl.BlockSpec(memory_space=pltpu.SMEM),
            pl.BlockSpec(memory_space=pltpu.SMEM),
            pl.BlockSpec((nb, C, H + 2 * PAD, W + 2 * PAD),
                         lambda m: (0, 0, 0, 0)),
        ],
        out_specs=[
            pl.BlockSpec((1, nb, HW), lambda m: (m, 0, 0)),
            pl.BlockSpec((1, nb, HW), lambda m: (m, 0, 0)),
        ],
        out_shape=[
            jax.ShapeDtypeStruct((M, nb, HW), jnp.int32),
            jax.ShapeDtypeStruct((M, nb, HW), jnp.float32),
        ],
    )(offs, thr, xp)


def _sc_body(idx_hbm, conf_hbm, table_hbm, out_hbm,
             idx_v, conf_v, gbuf, acc, gsem0, gsem1, lsem, osem):
    cid = lax.axis_index("c")
    sid = lax.axis_index("s")
    wid = sid * 2 + cid
    n = wid // 2
    rh = lax.rem(wid, 2)

    for m in range(M):
        pltpu.async_copy(idx_hbm.at[m, n], idx_v.at[m], lsem)
        pltpu.async_copy(conf_hbm.at[m, n], conf_v.at[m], lsem)
    for m in range(M):
        pltpu.make_async_copy(idx_hbm.at[m, n], idx_v.at[m], lsem).wait()
        pltpu.make_async_copy(conf_hbm.at[m, n], conf_v.at[m], lsem).wait()

    zero = jnp.zeros((16,), jnp.float32)
    for jj in range(WP):
        for q in range(4):
            acc[jj, pl.ds(16 * q, 16)] = zero

    def _issue(c, buf, sem):
        px0 = (c // 2) * W + lax.rem(c, 2) * 32
        for m in range(M):
            pltpu.async_copy(
                table_hbm.at[idx_v.at[m, pl.ds(px0, 32)]],
                gbuf.at[buf, m], sem)

    def _wait(c, buf, sem):
        px0 = (c // 2) * W + lax.rem(c, 2) * 32
        for m in range(M):
            pltpu.make_async_copy(
                table_hbm.at[idx_v.at[m, pl.ds(px0, 32)]],
                gbuf.at[buf, m], sem).wait()

    shl16 = jnp.full((16,), 16, jnp.int32)
    himask = jnp.full((16,), -65536, jnp.int32)

    def _compute(c, buf):
        px0 = (c // 2) * W + lax.rem(c, 2) * 32
        jbase = lax.rem(c, 2) * 16

        @pl.loop(0, 2)
        def _ph(ph):
            cvecs = [conf_v[m, pl.ds(px0 + 16 * ph, 16)] for m in range(M)]
            for p8 in range(8):
                accs = [None] * 8
                ri = 0
                for m in range(M):
                    for b in range(2):
                        cv = jnp.full((16,), cvecs[m][2 * p8 + b],
                                      jnp.float32)
                        row = 16 * ph + 2 * p8 + b
                        r0 = gbuf[buf, m, row, pl.ds(0, 16)]
                        r1 = gbuf[buf, m, row, pl.ds(16, 16)]
                        vals = (
                            plsc.bitcast(r0 << shl16, jnp.float32),
                            plsc.bitcast(r0 & himask, jnp.float32),
                            plsc.bitcast(r1 << shl16, jnp.float32),
                            plsc.bitcast(r1 & himask, jnp.float32),
                        )
                        g = (ri % 2) * 4
                        for q in range(4):
                            t = cv * vals[q]
                            accs[g + q] = (t if accs[g + q] is None
                                           else accs[g + q] + t)
                        ri += 1
                jj = jbase + 8 * ph + p8
                for q in range(4):
                    plsc.addupdate(acc.at[jj, pl.ds(16 * q, 16)],
                                   accs[q] + accs[4 + q])

    t0 = rh * HP
    _issue(2 * t0, 0, gsem0)

    @pl.loop(t0, t0 + HP)
    def _pair(t):
        c0 = 2 * t
        c1 = 2 * t + 1
        i = t // 2
        hh = lax.rem(t, 2)
        _issue(c1, 1, gsem1)
        _wait(c0, 0, gsem0)
        _compute(c0, 0)

        @pl.when(t < t0 + HP - 1)
        def _():
            _issue(c0 + 2, 0, gsem0)

        _wait(c1, 1, gsem1)
        _compute(c1, 1)

        @pl.when(hh == 1)
        def _():
            pltpu.sync_copy(acc, out_hbm.at[n, i])
            for jj in range(WP):
                for q in range(4):
                    acc[jj, pl.ds(16 * q, 16)] = zero


@functools.cache
def _sc_gather():
    mesh = plsc.VectorSubcoreMesh(core_axis_name="c", subcore_axis_name="s")
    return pl.kernel(
        _sc_body,
        out_type=jax.ShapeDtypeStruct((N // 2, HP, WP, DOUT), jnp.float32),
        mesh=mesh,
        scratch_types=[
            pltpu.VMEM((M, HW), jnp.int32),
            pltpu.VMEM((M, HW), jnp.float32),
            pltpu.VMEM((2, M, 32, DOUT // 2), jnp.int32),
            pltpu.VMEM((WP, DOUT), jnp.float32),
            pltpu.SemaphoreType.DMA,
            pltpu.SemaphoreType.DMA,
            pltpu.SemaphoreType.DMA,
            pltpu.SemaphoreType.DMA,
        ],
        compiler_params=pltpu.CompilerParams(use_tc_tiling_on_sc=False,
                                             needs_layout_passes=False),
    )


def _mm_body(a_ref, bt_ref, o_ref):
    o_ref[...] = jax.lax.dot_general(
        a_ref[...], bt_ref[...], (((1,), (1,)), ((), ())),
        preferred_element_type=jnp.float32)


def _stage3(flat, wt_t):
    return pl.pallas_call(
        _mm_body,
        out_shape=jax.ShapeDtypeStruct((N, NCLS), jnp.float32),
    )(flat, wt_t)


_DPERM = np.concatenate([np.arange(0, 32, 2), np.arange(1, 32, 2),
                         np.arange(32, 64, 2), np.arange(33, 64, 2)])


def kernel(x, thresholds, table, W_pred, b_pred, dy1, dx1, c1, dy2, dx2, c2):
    xp = jnp.pad(x, ((0, 0), (0, 0), (PAD, PAD), (PAD, PAD)))
    offs = jnp.stack([c1, dy1, dx1, c2, dy2, dx2], axis=-1).astype(jnp.int32)
    tpak = jax.lax.bitcast_convert_type(
        table.astype(jnp.bfloat16).reshape(M * NWORDS, DOUT // 2, 2),
        jnp.int32)
    halves = []
    for h0 in range(2):
        xph = xp[h0 * (N // 2):(h0 + 1) * (N // 2)]
        idx, conf = _stage1(xph, offs, thresholds)
        halves.append(_sc_gather()(idx, conf, tpak))
    pooled = jnp.concatenate(halves, axis=0)
    flat = pooled.reshape(N, HP * WP * DOUT)
    wt_t = W_pred.reshape(DOUT, HP * WP, NCLS)[_DPERM].transpose(
        2, 1, 0).reshape(NCLS, HP * WP * DOUT)
    return _stage3(flat, wt_t) + b_pred


# speedup vs baseline: 1.6177x; 1.0121x over previous
import functools

import numpy as np

import jax
import jax.numpy as jnp
from jax import lax
from jax.experimental import pallas as pl
from jax.experimental.pallas import tpu as pltpu
from jax.experimental.pallas import tpu_sc as plsc

M = 8
K = 10
L = 5
C = 3
H = 64
W = 64
N = 32
DOUT = 64
NCLS = 10
NWORDS = 2 ** K
PAD = L // 2
HW = H * W
HP = H // 2
WP = W // 2


def _stage1_body(nb, off_ref, thr_ref, xp_ref, idx_ref, conf_ref):
    m = pl.program_id(0)
    word = jnp.zeros((nb, H, W), jnp.int32)
    denom = jnp.ones((nb, H, W), jnp.float32)
    for k in range(K):
        c1k = off_ref[m, k, 0]
        dy1k = off_ref[m, k, 1]
        dx1k = off_ref[m, k, 2]
        c2k = off_ref[m, k, 3]
        dy2k = off_ref[m, k, 4]
        dx2k = off_ref[m, k, 5]
        v1 = xp_ref[:, c1k, pl.ds(dy1k, H), :]
        v2 = xp_ref[:, c2k, pl.ds(dy2k, H), :]
        p1 = pltpu.roll(v1, 68 - dx1k, axis=2)[:, :, :W]
        p2 = pltpu.roll(v2, 68 - dx2k, axis=2)[:, :, :W]
        z = (p1 - p2) - thr_ref[m, k]
        bit = z > 0.0
        word = word + jnp.where(bit, jnp.int32(1 << k), jnp.int32(0))
        denom = denom * (1.0 + jnp.exp(-jnp.abs(z)))
    idx_ref[0] = (word + m * NWORDS).reshape(nb, HW)
    conf_ref[0] = (0.25 / denom).reshape(nb, HW)


def _stage1(xp, offs, thr):
    nb = xp.shape[0]
    return pl.pallas_call(
        functools.partial(_stage1_body, nb),
        grid=(M,),
        in_specs=[
            pl.BlockSpec(memory_space=pltpu.SMEM),
            pl.BlockSpec(memory_space=pltpu.SMEM),
            pl.BlockSpec((nb, C, H + 2 * PAD, W + 2 * PAD),
                         lambda m: (0, 0, 0, 0)),
        ],
        out_specs=[
            pl.BlockSpec((1, nb, HW), lambda m: (m, 0, 0)),
            pl.BlockSpec((1, nb, HW), lambda m: (m, 0, 0)),
        ],
        out_shape=[
            jax.ShapeDtypeStruct((M, nb, HW), jnp.int32),
            jax.ShapeDtypeStruct((M, nb, HW), jnp.float32),
        ],
    )(offs, thr, xp)


def _sc_body(idx_hbm, conf_hbm, table_hbm, out_hbm,
             idx_v, conf_v, gbuf, acc, gsem0, gsem1, lsem, osem):
    cid = lax.axis_index("c")
    sid = lax.axis_index("s")
    wid = sid * 2 + cid
    n = wid // 2
    rh = lax.rem(wid, 2)

    for m in range(M):
        pltpu.async_copy(idx_hbm.at[m, n], idx_v.at[m], lsem)
        pltpu.async_copy(conf_hbm.at[m, n], conf_v.at[m], lsem)
    for m in range(M):
        pltpu.make_async_copy(idx_hbm.at[m, n], idx_v.at[m], lsem).wait()
        pltpu.make_async_copy(conf_hbm.at[m, n], conf_v.at[m], lsem).wait()

    zero = jnp.zeros((16,), jnp.float32)
    for jj in range(WP):
        for q in range(4):
            acc[jj, pl.ds(16 * q, 16)] = zero

    def _issue(c, buf, sem):
        px0 = (c // 2) * W + lax.rem(c, 2) * 32
        for m in range(M):
            pltpu.async_copy(
                table_hbm.at[idx_v.at[m, pl.ds(px0, 32)]],
                gbuf.at[buf, m], sem)

    def _wait(c, buf, sem):
        px0 = (c // 2) * W + lax.rem(c, 2) * 32
        for m in range(M):
            pltpu.make_async_copy(
                table_hbm.at[idx_v.at[m, pl.ds(px0, 32)]],
                gbuf.at[buf, m], sem).wait()

    shl16 = jnp.full((16,), 16, jnp.int32)
    himask = jnp.full((16,), -65536, jnp.int32)

    def _compute(c, buf):
        px0 = (c // 2) * W + lax.rem(c, 2) * 32
        jbase = lax.rem(c, 2) * 16

        @pl.loop(0, 2)
        def _ph(ph):
            cvecs = [conf_v[m, pl.ds(px0 + 16 * ph, 16)] for m in range(M)]
            for p8 in range(8):
                accs = [None] * 8
                ri = 0
                for m in range(M):
                    for b in range(2):
                        cv = jnp.full((16,), cvecs[m][2 * p8 + b],
                                      jnp.float32)
                        row = 16 * ph + 2 * p8 + b
                        r0 = gbuf[buf, m, row, pl.ds(0, 16)]
                        r1 = gbuf[buf, m, row, pl.ds(16, 16)]
                        vals = (
                            plsc.bitcast(r0 << shl16, jnp.float32),
                            plsc.bitcast(r0 & himask, jnp.float32),
                            plsc.bitcast(r1 << shl16, jnp.float32),
                            plsc.bitcast(r1 & himask, jnp.float32),
                        )
                        g = (ri % 2) * 4
                        for q in range(4):
                            t = cv * vals[q]
                            accs[g + q] = (t if accs[g + q] is None
                                           else accs[g + q] + t)
                        ri += 1
                jj = jbase + 8 * ph + p8
                for q in range(4):
                    plsc.addupdate(acc.at[jj, pl.ds(16 * q, 16)],
                                   accs[q] + accs[4 + q])

    t0 = rh * HP
    _issue(2 * t0, 0, gsem0)

    @pl.loop(t0, t0 + HP)
    def _pair(t):
        c0 = 2 * t
        c1 = 2 * t + 1
        i = t // 2
        hh = lax.rem(t, 2)
        _issue(c1, 1, gsem1)
        _wait(c0, 0, gsem0)
        _compute(c0, 0)

        @pl.when(t < t0 + HP - 1)
        def _():
            _issue(c0 + 2, 0, gsem0)

        _wait(c1, 1, gsem1)
        _compute(c1, 1)

        @pl.when(hh == 1)
        def _():
            pltpu.sync_copy(acc, out_hbm.at[n, i])
            for jj in range(WP):
                for q in range(4):
                    acc[jj, pl.ds(16 * q, 16)] = zero


@functools.cache
def _sc_gather():
    mesh = plsc.VectorSubcoreMesh(core_axis_name="c", subcore_axis_name="s")
    return pl.kernel(
        _sc_body,
        out_type=jax.ShapeDtypeStruct((N // 2, HP, WP, DOUT), jnp.float32),
        mesh=mesh,
        scratch_types=[
            pltpu.VMEM((M, HW), jnp.int32),
            pltpu.VMEM((M, HW), jnp.float32),
            pltpu.VMEM((2, M, 32, DOUT // 2), jnp.int32),
            pltpu.VMEM((WP, DOUT), jnp.float32),
            pltpu.SemaphoreType.DMA,
            pltpu.SemaphoreType.DMA,
            pltpu.SemaphoreType.DMA,
            pltpu.SemaphoreType.DMA,
        ],
        compiler_params=pltpu.CompilerParams(use_tc_tiling_on_sc=False,
                                             needs_layout_passes=False),
    )


def _mm_body(a_ref, bt_ref, o_ref):
    o_ref[...] = jax.lax.dot_general(
        a_ref[...], bt_ref[...], (((1,), (1,)), ((), ())),
        preferred_element_type=jnp.float32)


def _stage3(flat, wt_t):
    return pl.pallas_call(
        _mm_body,
        out_shape=jax.ShapeDtypeStruct((N, NCLS), jnp.float32),
    )(flat, wt_t)


_DPERM = np.concatenate([np.arange(0, 32, 2), np.arange(1, 32, 2),
                         np.arange(32, 64, 2), np.arange(33, 64, 2)])


def kernel(x, thresholds, table, W_pred, b_pred, dy1, dx1, c1, dy2, dx2, c2):
    xp = jnp.pad(x, ((0, 0), (0, 0), (PAD, PAD), (PAD, PAD)))
    offs = jnp.stack([c1, dy1, dx1, c2, dy2, dx2], axis=-1).astype(jnp.int32)
    tpak = jax.lax.bitcast_convert_type(
        table.astype(jnp.bfloat16).reshape(M * NWORDS, DOUT // 2, 2),
        jnp.int32)
    halves = []
    for h0 in range(2):
        xph = xp[h0 * (N // 2):(h0 + 1) * (N // 2)]
        idx, conf = _stage1(xph, offs, thresholds)
        halves.append(_sc_gather()(idx, conf, tpak))
    pooled = jnp.concatenate(halves, axis=0)
    flat = pooled.reshape(N, HP * WP * DOUT)
    wt_t = W_pred.reshape(DOUT, HP * WP, NCLS)[_DPERM].transpose(
        2, 1, 0).reshape(NCLS, HP * WP * DOUT)
    return _stage3(flat, wt_t) + b_pred


# speedup vs baseline: 1.7103x; 1.0573x over previous
import functools

import numpy as np

import jax
import jax.numpy as jnp
from jax import lax
from jax.experimental import pallas as pl
from jax.experimental.pallas import tpu as pltpu
from jax.experimental.pallas import tpu_sc as plsc

M = 8
K = 10
L = 5
C = 3
H = 64
W = 64
N = 32
DOUT = 64
NCLS = 10
NWORDS = 2 ** K
PAD = L // 2
HW = H * W
HP = H // 2
WP = W // 2


def _stage1_body(nb, off_ref, thr_ref, xp_ref, idx_ref, conf_ref):
    m = pl.program_id(0)
    word = jnp.zeros((nb, H, W), jnp.int32)
    denom = jnp.ones((nb, H, W), jnp.float32)
    for k in range(K):
        c1k = off_ref[m, k, 0]
        dy1k = off_ref[m, k, 1]
        dx1k = off_ref[m, k, 2]
        c2k = off_ref[m, k, 3]
        dy2k = off_ref[m, k, 4]
        dx2k = off_ref[m, k, 5]
        v1 = xp_ref[:, c1k, pl.ds(dy1k, H), :]
        v2 = xp_ref[:, c2k, pl.ds(dy2k, H), :]
        p1 = pltpu.roll(v1, 68 - dx1k, axis=2)[:, :, :W]
        p2 = pltpu.roll(v2, 68 - dx2k, axis=2)[:, :, :W]
        z = (p1 - p2) - thr_ref[m, k]
        bit = z > 0.0
        word = word + jnp.where(bit, jnp.int32(1 << k), jnp.int32(0))
        denom = denom * (1.0 + jnp.exp(-jnp.abs(z)))
    idx_ref[0] = (word + m * NWORDS).reshape(nb, HW)
    conf_ref[0] = (0.25 / denom).reshape(nb, HW)


def _stage1(xp, offs, thr):
    nb = xp.shape[0]
    return pl.pallas_call(
        functools.partial(_stage1_body, nb),
        grid=(M,),
        in_specs=[
            pl.BlockSpec(memory_space=pltpu.SMEM),
            pl.BlockSpec(memory_space=pltpu.SMEM),
            pl.BlockSpec((nb, C, H + 2 * PAD, W + 2 * PAD),
                         lambda m: (0, 0, 0, 0)),
        ],
        out_specs=[
            pl.BlockSpec((1, nb, HW), lambda m: (m, 0, 0)),
            pl.BlockSpec((1, nb, HW), lambda m: (m, 0, 0)),
        ],
        out_shape=[
            jax.ShapeDtypeStruct((M, nb, HW), jnp.int32),
            jax.ShapeDtypeStruct((M, nb, HW), jnp.float32),
        ],
    )(offs, thr, xp)


def _sc_body(idx_hbm, conf_hbm, table_hbm, out_hbm,
             idx_v, conf_v, gbuf, acc, gsem0, gsem1, lsem, osem):
    cid = lax.axis_index("c")
    sid = lax.axis_index("s")
    wid = sid * 2 + cid
    n = wid // 4
    rh = lax.rem(wid, 4)
    qpx = HW // 4

    for m in range(M):
        pltpu.async_copy(idx_hbm.at[m, n, pl.ds(rh * qpx, qpx)],
                         idx_v.at[m], lsem)
        pltpu.async_copy(conf_hbm.at[m, n, pl.ds(rh * qpx, qpx)],
                         conf_v.at[m], lsem)
    for m in range(M):
        pltpu.make_async_copy(idx_hbm.at[m, n, pl.ds(rh * qpx, qpx)],
                              idx_v.at[m], lsem).wait()
        pltpu.make_async_copy(conf_hbm.at[m, n, pl.ds(rh * qpx, qpx)],
                              conf_v.at[m], lsem).wait()

    zero = jnp.zeros((16,), jnp.float32)
    for jj in range(WP):
        for q in range(4):
            acc[jj, pl.ds(16 * q, 16)] = zero

    def _issue(c, buf, sem):
        px0 = (c // 2) * W + lax.rem(c, 2) * 32
        for m in range(M):
            pltpu.async_copy(
                table_hbm.at[idx_v.at[m, pl.ds(px0, 32)]],
                gbuf.at[buf, m], sem)

    def _wait(c, buf, sem):
        px0 = (c // 2) * W + lax.rem(c, 2) * 32
        for m in range(M):
            pltpu.make_async_copy(
                table_hbm.at[idx_v.at[m, pl.ds(px0, 32)]],
                gbuf.at[buf, m], sem).wait()

    shl16 = jnp.full((16,), 16, jnp.int32)
    himask = jnp.full((16,), -65536, jnp.int32)

    def _compute(c, buf):
        px0 = (c // 2) * W + lax.rem(c, 2) * 32
        jbase = lax.rem(c, 2) * 16

        @pl.loop(0, 2)
        def _ph(ph):
            cvecs = [conf_v[m, pl.ds(px0 + 16 * ph, 16)] for m in range(M)]
            for p8 in range(8):
                accs = [None] * 8
                ri = 0
                for m in range(M):
                    for b in range(2):
                        cv = jnp.full((16,), cvecs[m][2 * p8 + b],
                                      jnp.float32)
                        row = 16 * ph + 2 * p8 + b
                        r0 = gbuf[buf, m, row, pl.ds(0, 16)]
                        r1 = gbuf[buf, m, row, pl.ds(16, 16)]
                        vals = (
                            plsc.bitcast(r0 << shl16, jnp.float32),
                            plsc.bitcast(r0 & himask, jnp.float32),
                            plsc.bitcast(r1 << shl16, jnp.float32),
                            plsc.bitcast(r1 & himask, jnp.float32),
                        )
                        g = (ri % 2) * 4
                        for q in range(4):
                            t = cv * vals[q]
                            accs[g + q] = (t if accs[g + q] is None
                                           else accs[g + q] + t)
                        ri += 1
                jj = jbase + 8 * ph + p8
                for q in range(4):
                    plsc.addupdate(acc.at[jj, pl.ds(16 * q, 16)],
                                   accs[q] + accs[4 + q])

    nt = HP // 2
    _issue(0, 0, gsem0)

    @pl.loop(0, nt)
    def _pair(t):
        c0 = 2 * t
        c1 = 2 * t + 1
        i = rh * (HP // 4) + t // 2
        hh = lax.rem(t, 2)
        _issue(c1, 1, gsem1)
        _wait(c0, 0, gsem0)
        _compute(c0, 0)

        @pl.when(t < nt - 1)
        def _():
            _issue(c0 + 2, 0, gsem0)

        _wait(c1, 1, gsem1)
        _compute(c1, 1)

        @pl.when(hh == 1)
        def _():
            pltpu.sync_copy(acc, out_hbm.at[n, i])
            for jj in range(WP):
                for q in range(4):
                    acc[jj, pl.ds(16 * q, 16)] = zero


@functools.cache
def _sc_gather():
    mesh = plsc.VectorSubcoreMesh(core_axis_name="c", subcore_axis_name="s")
    return pl.kernel(
        _sc_body,
        out_type=jax.ShapeDtypeStruct((N // 4, HP, WP, DOUT), jnp.float32),
        mesh=mesh,
        scratch_types=[
            pltpu.VMEM((M, HW // 4), jnp.int32),
            pltpu.VMEM((M, HW // 4), jnp.float32),
            pltpu.VMEM((2, M, 32, DOUT // 2), jnp.int32),
            pltpu.VMEM((WP, DOUT), jnp.float32),
            pltpu.SemaphoreType.DMA,
            pltpu.SemaphoreType.DMA,
            pltpu.SemaphoreType.DMA,
            pltpu.SemaphoreType.DMA,
        ],
        compiler_params=pltpu.CompilerParams(use_tc_tiling_on_sc=False,
                                             needs_layout_passes=False),
    )


def _mm_body(a_ref, bt_ref, o_ref):
    o_ref[...] = jax.lax.dot_general(
        a_ref[...], bt_ref[...], (((1,), (1,)), ((), ())),
        preferred_element_type=jnp.float32)


def _stage3(flat, wt_t):
    return pl.pallas_call(
        _mm_body,
        out_shape=jax.ShapeDtypeStruct((N, NCLS), jnp.float32),
    )(flat, wt_t)


_DPERM = np.concatenate([np.arange(0, 32, 2), np.arange(1, 32, 2),
                         np.arange(32, 64, 2), np.arange(33, 64, 2)])


def kernel(x, thresholds, table, W_pred, b_pred, dy1, dx1, c1, dy2, dx2, c2):
    xp = jnp.pad(x, ((0, 0), (0, 0), (PAD, PAD), (PAD, PAD)))
    offs = jnp.stack([c1, dy1, dx1, c2, dy2, dx2], axis=-1).astype(jnp.int32)
    tpak = jax.lax.bitcast_convert_type(
        table.astype(jnp.bfloat16).reshape(M * NWORDS, DOUT // 2, 2),
        jnp.int32)
    parts = []
    for h0 in range(4):
        xph = xp[h0 * (N // 4):(h0 + 1) * (N // 4)]
        idx, conf = _stage1(xph, offs, thresholds)
        parts.append(_sc_gather()(idx, conf, tpak))
    pooled = jnp.concatenate(parts, axis=0)
    flat = pooled.reshape(N, HP * WP * DOUT)
    wt_t = W_pred.reshape(DOUT, HP * WP, NCLS)[_DPERM].transpose(
        2, 1, 0).reshape(NCLS, HP * WP * DOUT)
    return _stage3(flat, wt_t) + b_pred
